# Initial kernel scaffold; baseline (speedup 1.0000x reference)
import functools

import jax
import jax.numpy as jnp
from jax import lax
from jax.experimental import pallas as pl
from jax.experimental.pallas import tpu as pltpu
from jax.experimental.pallas import tpu_sc as plsc

N_ROWS = 160000
NUM_SEGMENTS = 10000
IN_DIM = 512
OUT_DIM = 256
FCHUNK = 128
BATCH = 80
ROWS_PER_TILE = N_ROWS // 16
NBATCH = ROWS_PER_TILE // BATCH
SEG_PER_TILE = NUM_SEGMENTS // 16


def _pool_body(x_hbm, idx_hbm, sums_hbm, cnt_hbm,
               idx_buf, row_buf, ones_buf, zbuf, zbuf_c, acc, cnt):
    core = lax.axis_index("c")
    sub = lax.axis_index("s")

    def _fill_z(t, _):
        i, k = t // 8, t % 8
        zbuf[i, pl.ds(k * 16, 16)] = jnp.zeros((16,), jnp.float32)
        return 0
    lax.fori_loop(0, 125 * 8, _fill_z, 0)

    def _fill_zc(i, _):
        zbuf_c[i, pl.ds(0, 16)] = jnp.zeros((16,), jnp.float32)
        return 0
    lax.fori_loop(0, SEG_PER_TILE, _fill_zc, 0)

    def _fill_ones(i, _):
        ones_buf[i, pl.ds(0, 16)] = jnp.ones((16,), jnp.float32)
        return 0
    lax.fori_loop(0, BATCH, _fill_ones, 0)

    seg0 = sub * SEG_PER_TILE

    for j in range(2):
        col0 = (2 * core + j) * FCHUNK

        for k in range(5):
            pltpu.sync_copy(zbuf, acc.at[pl.ds(seg0 + k * 125, 125)])
        if j == 0:
            @pl.when(core == 0)
            def _():
                pltpu.sync_copy(zbuf_c, cnt.at[pl.ds(seg0, SEG_PER_TILE)])
        plsc.subcore_barrier()

        def _batch(i, _):
            row0 = sub * ROWS_PER_TILE + i * BATCH
            pltpu.sync_copy(idx_hbm.at[pl.ds(row0, BATCH)], idx_buf)
            pltpu.sync_copy(x_hbm.at[pl.ds(row0, BATCH), pl.ds(col0, FCHUNK)],
                            row_buf)
            pltpu.sync_copy(row_buf, acc.at[idx_buf], add=True)
            if j == 0:
                @pl.when(core == 0)
                def _():
                    pltpu.sync_copy(ones_buf, cnt.at[idx_buf], add=True)
            return 0
        lax.fori_loop(0, NBATCH, _batch, 0)
        plsc.subcore_barrier()

        pltpu.sync_copy(acc.at[pl.ds(seg0, SEG_PER_TILE)],
                        sums_hbm.at[pl.ds(seg0, SEG_PER_TILE),
                                    pl.ds(col0, FCHUNK)])
        if j == 0:
            @pl.when(core == 0)
            def _():
                pltpu.sync_copy(cnt.at[pl.ds(seg0, SEG_PER_TILE)],
                                cnt_hbm.at[pl.ds(seg0, SEG_PER_TILE)])
            plsc.subcore_barrier()


_pool = functools.partial(
    pl.kernel,
    out_type=[
        jax.ShapeDtypeStruct((NUM_SEGMENTS, IN_DIM), jnp.float32),
        jax.ShapeDtypeStruct((NUM_SEGMENTS, 16), jnp.float32),
    ],
    mesh=plsc.VectorSubcoreMesh(core_axis_name="c", subcore_axis_name="s"),
    scratch_types=[
        pltpu.VMEM((BATCH,), jnp.int32),
        pltpu.VMEM((BATCH, FCHUNK), jnp.float32),
        pltpu.VMEM((BATCH, 16), jnp.float32),
        pltpu.VMEM((125, FCHUNK), jnp.float32),
        pltpu.VMEM((SEG_PER_TILE, 16), jnp.float32),
        pltpu.VMEM_SHARED((NUM_SEGMENTS, FCHUNK), jnp.float32),
        pltpu.VMEM_SHARED((NUM_SEGMENTS, 16), jnp.float32),
    ],
)(_pool_body)


def _proj_body(sums_ref, cnt_ref, w_ref, b_ref, out_ref):
    counts = cnt_ref[...][:, 0:1]
    pooled = sums_ref[...] / jnp.maximum(counts, 1.0)
    out_ref[...] = lax.dot_general(
        pooled, w_ref[...], (((1,), (1,)), ((), ())),
        preferred_element_type=jnp.float32) + b_ref[...]


def _project(sums, counts16, W, b2):
    bm = 1250
    grid = (NUM_SEGMENTS // bm,)
    return pl.pallas_call(
        _proj_body,
        grid=grid,
        in_specs=[
            pl.BlockSpec((bm, IN_DIM), lambda i: (i, 0)),
            pl.BlockSpec((bm, 16), lambda i: (i, 0)),
            pl.BlockSpec((OUT_DIM, IN_DIM), lambda i: (0, 0)),
            pl.BlockSpec((1, OUT_DIM), lambda i: (0, 0)),
        ],
        out_specs=pl.BlockSpec((bm, OUT_DIM), lambda i: (i, 0)),
        out_shape=jax.ShapeDtypeStruct((NUM_SEGMENTS, OUT_DIM), jnp.float32),
    )(sums, counts16, W, b2)


def kernel(io_concat_embed, scatter_idx, W, b):
    idx = scatter_idx.astype(jnp.int32)
    sums, counts16 = _pool(io_concat_embed, idx)
    return _project(sums, counts16, W, b.reshape(1, OUT_DIM))


# speedup vs baseline: 1.4600x; 1.4600x over previous
import functools

import jax
import jax.numpy as jnp
from jax import lax
from jax.experimental import pallas as pl
from jax.experimental.pallas import tpu as pltpu
from jax.experimental.pallas import tpu_sc as plsc

N_ROWS = 160000
NUM_SEGMENTS = 10000
IN_DIM = 512
OUT_DIM = 256
FCHUNK = 64
BATCH = 80
ROWS_PER_TILE = N_ROWS // 16
NBATCH = ROWS_PER_TILE // BATCH
SEG_SLICE = 624
SEG_SLICE_LAST = NUM_SEGMENTS - 15 * SEG_SLICE


def _pool_body(x_hbm, idx_hbm, sums_hbm, cnt_hbm,
               idx_buf, row_buf, ones_buf, zbuf, zbuf_c, acc, cnt):
    core = lax.axis_index("c")
    sub = lax.axis_index("s")

    def _fill_z(t, _):
        i, k = t // (FCHUNK // 16), t % (FCHUNK // 16)
        zbuf[i, pl.ds(k * 16, 16)] = jnp.zeros((16,), jnp.float32)
        return 0
    lax.fori_loop(0, SEG_SLICE_LAST * (FCHUNK // 16), _fill_z, 0)

    def _fill_zc(i, _):
        zbuf_c[i, pl.ds(0, 16)] = jnp.zeros((16,), jnp.float32)
        return 0
    lax.fori_loop(0, SEG_SLICE_LAST, _fill_zc, 0)

    def _fill_ones(i, _):
        ones_buf[i, pl.ds(0, 16)] = jnp.ones((16,), jnp.float32)
        return 0
    lax.fori_loop(0, BATCH, _fill_ones, 0)

    seg0 = sub * SEG_SLICE
    is_last = sub == 15

    for j in range(IN_DIM // FCHUNK // 2):
        col0 = ((IN_DIM // FCHUNK // 2) * core + j) * FCHUNK

        @pl.when(jnp.logical_not(is_last))
        def _():
            pltpu.sync_copy(zbuf.at[pl.ds(0, SEG_SLICE)],
                            acc.at[pl.ds(seg0, SEG_SLICE)])
        @pl.when(is_last)
        def _():
            pltpu.sync_copy(zbuf, acc.at[pl.ds(seg0, SEG_SLICE_LAST)])
        if j == 0:
            @pl.when(jnp.logical_and(core == 0, jnp.logical_not(is_last)))
            def _():
                pltpu.sync_copy(zbuf_c.at[pl.ds(0, SEG_SLICE)],
                                cnt.at[pl.ds(seg0, SEG_SLICE)])
            @pl.when(jnp.logical_and(core == 0, is_last))
            def _():
                pltpu.sync_copy(zbuf_c, cnt.at[pl.ds(seg0, SEG_SLICE_LAST)])
        plsc.subcore_barrier()

        def _batch(i, _):
            row0 = sub * ROWS_PER_TILE + i * BATCH
            pltpu.sync_copy(idx_hbm.at[pl.ds(row0, BATCH)], idx_buf)
            pltpu.sync_copy(x_hbm.at[pl.ds(row0, BATCH), pl.ds(col0, FCHUNK)],
                            row_buf)
            pltpu.sync_copy(row_buf, acc.at[idx_buf], add=True)
            if j == 0:
                @pl.when(core == 0)
                def _():
                    pltpu.sync_copy(ones_buf, cnt.at[idx_buf], add=True)
            return 0
        lax.fori_loop(0, NBATCH, _batch, 0)
        plsc.subcore_barrier()

        @pl.when(jnp.logical_not(is_last))
        def _():
            pltpu.sync_copy(acc.at[pl.ds(seg0, SEG_SLICE)],
                            sums_hbm.at[pl.ds(seg0, SEG_SLICE),
                                        pl.ds(col0, FCHUNK)])
        @pl.when(is_last)
        def _():
            pltpu.sync_copy(acc.at[pl.ds(seg0, SEG_SLICE_LAST)],
                            sums_hbm.at[pl.ds(seg0, SEG_SLICE_LAST),
                                        pl.ds(col0, FCHUNK)])
        if j == 0:
            @pl.when(jnp.logical_and(core == 0, jnp.logical_not(is_last)))
            def _():
                pltpu.sync_copy(cnt.at[pl.ds(seg0, SEG_SLICE)],
                                cnt_hbm.at[pl.ds(seg0, SEG_SLICE)])
            @pl.when(jnp.logical_and(core == 0, is_last))
            def _():
                pltpu.sync_copy(cnt.at[pl.ds(seg0, SEG_SLICE_LAST)],
                                cnt_hbm.at[pl.ds(seg0, SEG_SLICE_LAST)])
            plsc.subcore_barrier()


_pool = functools.partial(
    pl.kernel,
    out_type=[
        jax.ShapeDtypeStruct((NUM_SEGMENTS, IN_DIM), jnp.float32),
        jax.ShapeDtypeStruct((NUM_SEGMENTS, 16), jnp.float32),
    ],
    mesh=plsc.VectorSubcoreMesh(core_axis_name="c", subcore_axis_name="s"),
    compiler_params=pltpu.CompilerParams(use_tc_tiling_on_sc=False),
    scratch_types=[
        pltpu.VMEM((BATCH,), jnp.int32),
        pltpu.VMEM((BATCH, FCHUNK), jnp.float32),
        pltpu.VMEM((BATCH, 16), jnp.float32),
        pltpu.VMEM((SEG_SLICE_LAST, FCHUNK), jnp.float32),
        pltpu.VMEM((SEG_SLICE_LAST, 16), jnp.float32),
        pltpu.VMEM_SHARED((NUM_SEGMENTS, FCHUNK), jnp.float32),
        pltpu.VMEM_SHARED((NUM_SEGMENTS, 16), jnp.float32),
    ],
)(_pool_body)


def _proj_body(sums_ref, cnt_ref, w_ref, b_ref, out_ref):
    counts = cnt_ref[...][:, 0:1]
    pooled = sums_ref[...] / jnp.maximum(counts, 1.0)
    out_ref[...] = lax.dot_general(
        pooled, w_ref[...], (((1,), (1,)), ((), ())),
        preferred_element_type=jnp.float32) + b_ref[...]


def _project(sums, counts16, W, b2):
    bm = 1000
    grid = (NUM_SEGMENTS // bm,)
    return pl.pallas_call(
        _proj_body,
        grid=grid,
        in_specs=[
            pl.BlockSpec((bm, IN_DIM), lambda i: (i, 0)),
            pl.BlockSpec((bm, 16), lambda i: (i, 0)),
            pl.BlockSpec((OUT_DIM, IN_DIM), lambda i: (0, 0)),
            pl.BlockSpec((1, OUT_DIM), lambda i: (0, 0)),
        ],
        out_specs=pl.BlockSpec((bm, OUT_DIM), lambda i: (i, 0)),
        out_shape=jax.ShapeDtypeStruct((NUM_SEGMENTS, OUT_DIM), jnp.float32),
    )(sums, counts16, W, b2)


def kernel(io_concat_embed, scatter_idx, W, b):
    idx = scatter_idx.astype(jnp.int32)
    sums, counts16 = _pool(io_concat_embed, idx)
    return _project(sums, counts16, W, b.reshape(1, OUT_DIM))


# speedup vs baseline: 2.8168x; 1.9293x over previous
import functools

import jax
import jax.numpy as jnp
from jax import lax
from jax.experimental import pallas as pl
from jax.experimental.pallas import tpu as pltpu
from jax.experimental.pallas import tpu_sc as plsc

N_ROWS = 160000
NUM_SEGMENTS = 10000
IN_DIM = 512
OUT_DIM = 256
FCHUNK = 64
NCHUNK = IN_DIM // FCHUNK // 2
BATCH = 128
ROWS_PER_TILE = N_ROWS // 16
NFULL = ROWS_PER_TILE // BATCH
TAIL = ROWS_PER_TILE - NFULL * BATCH
GSZ = 3
NG = NFULL // GSZ
SEG_SLICE = 624
SEG_SLICE_LAST = NUM_SEGMENTS - 15 * SEG_SLICE
ZROWS = 160


def _pool_body(x_hbm, idx_hbm, sums_hbm, cnt_hbm,
               idx_all, bank_a, bank_b, tail_buf, ones_buf, zbuf,
               acc, sem_la, sem_lb, sem_sa, sem_sb, sem_c):
    core = lax.axis_index("c")
    sub = lax.axis_index("s")

    pltpu.sync_copy(idx_hbm.at[pl.ds(sub * ROWS_PER_TILE, ROWS_PER_TILE)],
                    idx_all)

    def _fill_z(t, _):
        i, k = t // (FCHUNK // 16), t % (FCHUNK // 16)
        zbuf[i, pl.ds(k * 16, 16)] = jnp.zeros((16,), jnp.float32)
        return 0
    lax.fori_loop(0, ZROWS * (FCHUNK // 16), _fill_z, 0)

    def _fill_ones(t, _):
        i, k = t // (FCHUNK // 16), t % (FCHUNK // 16)
        ones_buf[i, pl.ds(k * 16, 16)] = jnp.ones((16,), jnp.float32)
        return 0
    lax.fori_loop(0, BATCH * (FCHUNK // 16), _fill_ones, 0)

    seg0 = sub * SEG_SLICE
    is_last = sub == 15

    def load_desc(g, k, bank, sem, col0):
        row0 = sub * ROWS_PER_TILE + (g * GSZ + k) * BATCH
        return pltpu.make_async_copy(
            x_hbm.at[pl.ds(row0, BATCH), pl.ds(col0, FCHUNK)], bank.at[k], sem)

    def scat_desc(g, k, bank, sem):
        b = g * GSZ + k
        return pltpu.make_async_copy(
            bank.at[k], acc.at[idx_all.at[pl.ds(b * BATCH, BATCH)]], sem)

    @pl.when(core == 1)
    def _():
        for k in range(4):
            @pl.when(jnp.logical_not(is_last))
            def _():
                pltpu.sync_copy(zbuf.at[pl.ds(0, 156)],
                                acc.at[pl.ds(seg0 + k * 156, 156)])
            @pl.when(is_last)
            def _():
                pltpu.sync_copy(zbuf, acc.at[pl.ds(seg0 + k * ZROWS, ZROWS)])
        plsc.subcore_barrier()

        def cnt_desc(b):
            return pltpu.make_async_copy(
                ones_buf, acc.at[idx_all.at[pl.ds(b * BATCH, BATCH)]], sem_c)

        def _cgroup(i, _):
            for k in range(6):
                pltpu.async_copy(
                    ones_buf, acc.at[idx_all.at[pl.ds((6 * i + k) * BATCH,
                                                      BATCH)]],
                    sem_c, add=True)
            @pl.when(i > 0)
            def _():
                for k in range(6):
                    cnt_desc(6 * (i - 1) + k).wait()
            return 0
        lax.fori_loop(0, NFULL // 6, _cgroup, 0)
        for k in range(6):
            cnt_desc(NFULL - 6 + k).wait()
        pltpu.sync_copy(ones_buf.at[pl.ds(0, TAIL)],
                        acc.at[idx_all.at[pl.ds(NFULL * BATCH, TAIL)]],
                        add=True)
        plsc.subcore_barrier()

        @pl.when(jnp.logical_not(is_last))
        def _():
            pltpu.sync_copy(acc.at[pl.ds(seg0, SEG_SLICE), pl.ds(0, 16)],
                            cnt_hbm.at[pl.ds(seg0, SEG_SLICE)])
        @pl.when(is_last)
        def _():
            pltpu.sync_copy(acc.at[pl.ds(seg0, SEG_SLICE_LAST), pl.ds(0, 16)],
                            cnt_hbm.at[pl.ds(seg0, SEG_SLICE_LAST)])

    for j in range(NCHUNK):
        col0 = (NCHUNK * core + j) * FCHUNK

        for k in range(4):
            @pl.when(jnp.logical_not(is_last))
            def _():
                pltpu.sync_copy(zbuf.at[pl.ds(0, 156)],
                                acc.at[pl.ds(seg0 + k * 156, 156)])
            @pl.when(is_last)
            def _():
                pltpu.sync_copy(zbuf, acc.at[pl.ds(seg0 + k * ZROWS, ZROWS)])
        plsc.subcore_barrier()

        for k in range(GSZ):
            load_desc(0, k, bank_a, sem_la, col0).start()

        def _pair(p, _):
            ga, gb = 2 * p, 2 * p + 1
            for k in range(GSZ):
                load_desc(ga, k, bank_a, sem_la, col0).wait()
            for k in range(GSZ):
                pltpu.async_copy(bank_a.at[k],
                                 acc.at[idx_all.at[pl.ds((ga * GSZ + k) * BATCH,
                                                         BATCH)]],
                                 sem_sa, add=True)
            @pl.when(p > 0)
            def _():
                for k in range(GSZ):
                    scat_desc(gb - 2, k, bank_b, sem_sb).wait()
            for k in range(GSZ):
                load_desc(gb, k, bank_b, sem_lb, col0).start()
            for k in range(GSZ):
                load_desc(gb, k, bank_b, sem_lb, col0).wait()
            for k in range(GSZ):
                pltpu.async_copy(bank_b.at[k],
                                 acc.at[idx_all.at[pl.ds((gb * GSZ + k) * BATCH,
                                                         BATCH)]],
                                 sem_sb, add=True)
            for k in range(GSZ):
                scat_desc(ga, k, bank_a, sem_sa).wait()
            @pl.when(p < (NG // 2 - 1))
            def _():
                for k in range(GSZ):
                    load_desc(ga + 2, k, bank_a, sem_la, col0).start()
            return 0
        lax.fori_loop(0, NG // 2, _pair, 0)
        for k in range(GSZ):
            scat_desc(NG - 1, k, bank_b, sem_sb).wait()

        rowt = sub * ROWS_PER_TILE + NFULL * BATCH
        pltpu.sync_copy(x_hbm.at[pl.ds(rowt, TAIL), pl.ds(col0, FCHUNK)],
                        tail_buf)
        pltpu.sync_copy(tail_buf,
                        acc.at[idx_all.at[pl.ds(NFULL * BATCH, TAIL)]],
                        add=True)
        plsc.subcore_barrier()

        @pl.when(jnp.logical_not(is_last))
        def _():
            pltpu.sync_copy(acc.at[pl.ds(seg0, SEG_SLICE)],
                            sums_hbm.at[pl.ds(seg0, SEG_SLICE),
                                        pl.ds(col0, FCHUNK)])
        @pl.when(is_last)
        def _():
            pltpu.sync_copy(acc.at[pl.ds(seg0, SEG_SLICE_LAST)],
                            sums_hbm.at[pl.ds(seg0, SEG_SLICE_LAST),
                                        pl.ds(col0, FCHUNK)])


_pool = functools.partial(
    pl.kernel,
    out_type=[
        jax.ShapeDtypeStruct((NUM_SEGMENTS, IN_DIM), jnp.float32),
        jax.ShapeDtypeStruct((NUM_SEGMENTS, 16), jnp.float32),
    ],
    mesh=plsc.VectorSubcoreMesh(core_axis_name="c", subcore_axis_name="s"),
    compiler_params=pltpu.CompilerParams(use_tc_tiling_on_sc=False),
    scratch_types=[
        pltpu.VMEM((ROWS_PER_TILE,), jnp.int32),
        pltpu.VMEM((GSZ, BATCH, FCHUNK), jnp.float32),
        pltpu.VMEM((GSZ, BATCH, FCHUNK), jnp.float32),
        pltpu.VMEM((TAIL, FCHUNK), jnp.float32),
        pltpu.VMEM((BATCH, FCHUNK), jnp.float32),
        pltpu.VMEM((ZROWS, FCHUNK), jnp.float32),
        pltpu.VMEM_SHARED((NUM_SEGMENTS, FCHUNK), jnp.float32),
        pltpu.SemaphoreType.DMA,
        pltpu.SemaphoreType.DMA,
        pltpu.SemaphoreType.DMA,
        pltpu.SemaphoreType.DMA,
        pltpu.SemaphoreType.DMA,
    ],
)(_pool_body)


def _proj_body(sums_ref, cnt_ref, w_ref, b_ref, out_ref):
    counts = cnt_ref[...][:, 0:1]
    pooled = sums_ref[...] / jnp.maximum(counts, 1.0)
    out_ref[...] = lax.dot_general(
        pooled, w_ref[...], (((1,), (1,)), ((), ())),
        preferred_element_type=jnp.float32) + b_ref[...]


def _project(sums, counts16, W, b2):
    bm = 1000
    grid = (NUM_SEGMENTS // bm,)
    return pl.pallas_call(
        _proj_body,
        grid=grid,
        in_specs=[
            pl.BlockSpec((bm, IN_DIM), lambda i: (i, 0)),
            pl.BlockSpec((bm, 16), lambda i: (i, 0)),
            pl.BlockSpec((OUT_DIM, IN_DIM), lambda i: (0, 0)),
            pl.BlockSpec((1, OUT_DIM), lambda i: (0, 0)),
        ],
        out_specs=pl.BlockSpec((bm, OUT_DIM), lambda i: (i, 0)),
        out_shape=jax.ShapeDtypeStruct((NUM_SEGMENTS, OUT_DIM), jnp.float32),
    )(sums, counts16, W, b2)


def kernel(io_concat_embed, scatter_idx, W, b):
    idx = scatter_idx.astype(jnp.int32)
    sums, counts16 = _pool(io_concat_embed, idx)
    return _project(sums, counts16, W, b.reshape(1, OUT_DIM))


# speedup vs baseline: 4.3192x; 1.5334x over previous
import functools

import jax
import jax.numpy as jnp
from jax import lax
from jax.experimental import pallas as pl
from jax.experimental.pallas import tpu as pltpu
from jax.experimental.pallas import tpu_sc as plsc

N_ROWS = 160000
NUM_SEGMENTS = 10000
IN_DIM = 512
OUT_DIM = 256
FCHUNK = 128
NCHUNK = IN_DIM // FCHUNK // 2
BATCH = 128
SLABS = BATCH // 8
ROWS_PER_TILE = N_ROWS // 16
A_PER_TILE = ROWS_PER_TILE // 8
NFULL = ROWS_PER_TILE // BATCH
TAIL = ROWS_PER_TILE - NFULL * BATCH
SEG_SLICE = 624
SEG_SLICE_LAST = NUM_SEGMENTS - 15 * SEG_SLICE


def _pool_body(x_hbm, idx_hbm, sums_hbm, cnt_hbm,
               idx_all, buf_a, buf_b, tail_buf,
               acc, sem_la, sem_lb, sem_sa, sem_sb, sem_c):
    core = lax.axis_index("c")
    sub = lax.axis_index("s")

    pltpu.sync_copy(idx_hbm.at[pl.ds(sub * ROWS_PER_TILE, ROWS_PER_TILE)],
                    idx_all)

    seg0 = sub * SEG_SLICE
    is_last = sub == 15

    def _fill(buf, value, nwords):
        def _f(t, _):
            i = t // (FCHUNK // 16)
            k = t % (FCHUNK // 16)
            buf[i, pl.ds(k * 16, 16)] = value
            return 0
        lax.fori_loop(0, nwords // 16, _f, 0)

    def _zero_acc():
        zsrc = buf_a

        @pl.when(jnp.logical_not(is_last))
        def _():
            for k in range(4):
                pltpu.sync_copy(zsrc.at[pl.ds(0, 125)],
                                acc.at[pl.ds(seg0 + k * 125, 125)])
            pltpu.sync_copy(zsrc.at[pl.ds(0, 124)],
                            acc.at[pl.ds(seg0 + 500, 124)])
        @pl.when(is_last)
        def _():
            for k in range(5):
                pltpu.sync_copy(zsrc, acc.at[pl.ds(seg0 + k * BATCH, BATCH)])

    def load_desc(b, buf, sem, d, k):
        a0 = sub * A_PER_TILE + b * SLABS
        return pltpu.make_async_copy(x_hbm.at[a0 + k, d],
                                     buf.at[pl.ds(k * 8, 8)], sem)

    def load_start(b, buf, sem, d):
        for k in range(SLABS):
            load_desc(b, buf, sem, d, k).start()

    def load_wait(b, buf, sem, d):
        for k in range(SLABS):
            load_desc(b, buf, sem, d, k).wait()

    def scat_desc(b, buf, sem):
        return pltpu.make_async_copy(
            buf, acc.at[idx_all.at[pl.ds(b * BATCH, BATCH)]], sem)

    def _scatter_start(b, buf, sem):
        pltpu.async_copy(buf, acc.at[idx_all.at[pl.ds(b * BATCH, BATCH)]],
                         sem, add=True)

    @pl.when(core == 1)
    def _():
        _fill(buf_a, jnp.zeros((16,), jnp.float32), BATCH * FCHUNK)
        _zero_acc()
        _fill(buf_b, jnp.ones((16,), jnp.float32), BATCH * FCHUNK)
        plsc.subcore_barrier()

        ones = buf_b

        def cnt_desc(b):
            return pltpu.make_async_copy(
                ones, acc.at[idx_all.at[pl.ds(b * BATCH, BATCH)]], sem_c)

        def _cgroup(i, _):
            for k in range(6):
                pltpu.async_copy(
                    ones, acc.at[idx_all.at[pl.ds((6 * i + k) * BATCH, BATCH)]],
                    sem_c, add=True)
            @pl.when(i > 0)
            def _():
                for k in range(6):
                    cnt_desc(6 * (i - 1) + k).wait()
            return 0
        lax.fori_loop(0, NFULL // 6, _cgroup, 0)
        for k in range(6):
            cnt_desc(NFULL - 6 + k).wait()
        pltpu.sync_copy(ones.at[pl.ds(0, TAIL)],
                        acc.at[idx_all.at[pl.ds(NFULL * BATCH, TAIL)]],
                        add=True)
        plsc.subcore_barrier()

        @pl.when(jnp.logical_not(is_last))
        def _():
            pltpu.sync_copy(acc.at[pl.ds(seg0, SEG_SLICE), pl.ds(0, 16)],
                            cnt_hbm.at[pl.ds(seg0, SEG_SLICE)])
        @pl.when(is_last)
        def _():
            pltpu.sync_copy(acc.at[pl.ds(seg0, SEG_SLICE_LAST), pl.ds(0, 16)],
                            cnt_hbm.at[pl.ds(seg0, SEG_SLICE_LAST)])

    for j in range(NCHUNK):
        d = NCHUNK * core + j

        _fill(buf_a, jnp.zeros((16,), jnp.float32), BATCH * FCHUNK)
        _zero_acc()
        plsc.subcore_barrier()

        load_start(0, buf_a, sem_la, d)

        def _pair(p, _):
            ba, bb = 2 * p, 2 * p + 1
            load_wait(ba, buf_a, sem_la, d)
            _scatter_start(ba, buf_a, sem_sa)
            @pl.when(p > 0)
            def _():
                scat_desc(bb - 2, buf_b, sem_sb).wait()
            load_start(bb, buf_b, sem_lb, d)
            load_wait(bb, buf_b, sem_lb, d)
            _scatter_start(bb, buf_b, sem_sb)
            scat_desc(ba, buf_a, sem_sa).wait()
            @pl.when(p < (NFULL // 2 - 1))
            def _():
                load_start(ba + 2, buf_a, sem_la, d)
            return 0
        lax.fori_loop(0, NFULL // 2, _pair, 0)
        scat_desc(NFULL - 1, buf_b, sem_sb).wait()

        a_t = sub * A_PER_TILE + NFULL * SLABS
        pltpu.sync_copy(x_hbm.at[a_t, d], tail_buf.at[pl.ds(0, 8)])
        pltpu.sync_copy(x_hbm.at[a_t + 1, d], tail_buf.at[pl.ds(8, 8)])
        pltpu.sync_copy(tail_buf,
                        acc.at[idx_all.at[pl.ds(NFULL * BATCH, TAIL)]],
                        add=True)
        plsc.subcore_barrier()

        nslab = jnp.where(is_last, SEG_SLICE_LAST // 8, SEG_SLICE // 8)

        def _co_desc(k):
            return pltpu.make_async_copy(
                acc.at[pl.ds(seg0 + k * 8, 8)],
                sums_hbm.at[sub * (SEG_SLICE // 8) + k, d], sem_c)

        def _co_fire(k, _):
            _co_desc(k).start()
            return 0
        def _co_drain(k, _):
            _co_desc(k).wait()
            return 0
        lax.fori_loop(0, nslab, _co_fire, 0)
        lax.fori_loop(0, nslab, _co_drain, 0)


_pool = functools.partial(
    pl.kernel,
    out_type=[
        jax.ShapeDtypeStruct((NUM_SEGMENTS // 8, IN_DIM // FCHUNK, 8, FCHUNK),
                             jnp.float32),
        jax.ShapeDtypeStruct((NUM_SEGMENTS, 16), jnp.float32),
    ],
    mesh=plsc.VectorSubcoreMesh(core_axis_name="c", subcore_axis_name="s"),
    compiler_params=pltpu.CompilerParams(use_tc_tiling_on_sc=False),
    scratch_types=[
        pltpu.VMEM((ROWS_PER_TILE,), jnp.int32),
        pltpu.VMEM((BATCH, FCHUNK), jnp.float32),
        pltpu.VMEM((BATCH, FCHUNK), jnp.float32),
        pltpu.VMEM((TAIL, FCHUNK), jnp.float32),
        pltpu.VMEM_SHARED((NUM_SEGMENTS, FCHUNK), jnp.float32),
        pltpu.SemaphoreType.DMA,
        pltpu.SemaphoreType.DMA,
        pltpu.SemaphoreType.DMA,
        pltpu.SemaphoreType.DMA,
        pltpu.SemaphoreType.DMA,
    ],
)(_pool_body)


def _proj_body(sums_ref, cnt_ref, w_ref, b_ref, out_ref):
    counts = cnt_ref[...][:, 0:1]
    pooled = sums_ref[...] / jnp.maximum(counts, 1.0)
    out_ref[...] = lax.dot_general(
        pooled, w_ref[...], (((1,), (1,)), ((), ())),
        preferred_element_type=jnp.float32) + b_ref[...]


def _project(sums, counts16, W, b2):
    bm = 1000
    grid = (NUM_SEGMENTS // bm,)
    return pl.pallas_call(
        _proj_body,
        grid=grid,
        in_specs=[
            pl.BlockSpec((bm, IN_DIM), lambda i: (i, 0)),
            pl.BlockSpec((bm, 16), lambda i: (i, 0)),
            pl.BlockSpec((OUT_DIM, IN_DIM), lambda i: (0, 0)),
            pl.BlockSpec((1, OUT_DIM), lambda i: (0, 0)),
        ],
        out_specs=pl.BlockSpec((bm, OUT_DIM), lambda i: (i, 0)),
        out_shape=jax.ShapeDtypeStruct((NUM_SEGMENTS, OUT_DIM), jnp.float32),
    )(sums, counts16, W, b2)


def kernel(io_concat_embed, scatter_idx, W, b):
    idx = scatter_idx.astype(jnp.int32)
    x4 = jnp.transpose(
        jnp.reshape(io_concat_embed,
                    (N_ROWS // 8, 8, IN_DIM // FCHUNK, FCHUNK)),
        (0, 2, 1, 3))
    sums4, counts16 = _pool(x4, idx)
    sums = jnp.reshape(jnp.transpose(sums4, (0, 2, 1, 3)),
                       (NUM_SEGMENTS, IN_DIM))
    return _project(sums, counts16, W, b.reshape(1, OUT_DIM))


# speedup vs baseline: 4.8774x; 1.1292x over previous
import functools

import jax
import jax.numpy as jnp
from jax import lax
from jax.experimental import pallas as pl
from jax.experimental.pallas import tpu as pltpu
from jax.experimental.pallas import tpu_sc as plsc

N_ROWS = 160000
NUM_SEGMENTS = 10000
IN_DIM = 512
OUT_DIM = 256
FCHUNK = 128
NCHUNK = IN_DIM // FCHUNK // 2
BATCH = 64
SLABS = BATCH // 8
ROWS_PER_TILE = N_ROWS // 16
A_PER_TILE = ROWS_PER_TILE // 8
NFULL = ROWS_PER_TILE // BATCH
TAIL = ROWS_PER_TILE - NFULL * BATCH
CROWS = ROWS_PER_TILE // 2
CFULL = CROWS // BATCH
CTAIL = CROWS - CFULL * BATCH
SEG_SLICE = 624
SEG_SLICE_LAST = NUM_SEGMENTS - 15 * SEG_SLICE


def _pool_body(x_hbm, idx_hbm, sums_hbm, cnt_hbm,
               idx_all, b0, b1, b2, b3, tail_buf, acc,
               l0, l1, l2, l3, s0, s1, s2, s3, sem_c):
    core = lax.axis_index("c")
    sub = lax.axis_index("s")
    bufs = (b0, b1, b2, b3)
    lsems = (l0, l1, l2, l3)
    ssems = (s0, s1, s2, s3)

    pltpu.sync_copy(idx_hbm.at[pl.ds(sub * ROWS_PER_TILE, ROWS_PER_TILE)],
                    idx_all)

    seg0 = sub * SEG_SLICE
    is_last = sub == 15

    def _fill(buf, value):
        def _f(t, _):
            i = t // (FCHUNK // 16)
            k = t % (FCHUNK // 16)
            buf[i, pl.ds(k * 16, 16)] = value
            return 0
        lax.fori_loop(0, BATCH * FCHUNK // 16, _f, 0)

    def _zero_acc():
        @pl.when(jnp.logical_not(is_last))
        def _():
            for k in range(9):
                pltpu.sync_copy(b0, acc.at[pl.ds(seg0 + k * BATCH, BATCH)])
            pltpu.sync_copy(b0.at[pl.ds(0, 48)],
                            acc.at[pl.ds(seg0 + 9 * BATCH, 48)])
        @pl.when(is_last)
        def _():
            for k in range(10):
                pltpu.sync_copy(b0, acc.at[pl.ds(seg0 + k * BATCH, BATCH)])

    def load_desc(b, buf, sem, d, k):
        a0 = sub * A_PER_TILE + b * SLABS
        return pltpu.make_async_copy(x_hbm.at[a0 + k, d],
                                     buf.at[pl.ds(k * 8, 8)], sem)

    def load_start(b, buf, sem, d):
        for k in range(SLABS):
            load_desc(b, buf, sem, d, k).start()

    def load_wait(b, buf, sem, d):
        for k in range(SLABS):
            load_desc(b, buf, sem, d, k).wait()

    def scat_desc(b, buf, sem):
        return pltpu.make_async_copy(
            buf, acc.at[idx_all.at[pl.ds(b * BATCH, BATCH)]], sem)

    def scat_start(b, buf, sem):
        pltpu.async_copy(buf, acc.at[idx_all.at[pl.ds(b * BATCH, BATCH)]],
                         sem, add=True)

    _fill(b0, jnp.zeros((16,), jnp.float32))
    _zero_acc()
    _fill(b1, jnp.ones((16,), jnp.float32))
    plsc.subcore_barrier()

    r0 = core * CROWS

    def cnt_desc(b):
        return pltpu.make_async_copy(
            b1, acc.at[idx_all.at[pl.ds(r0 + b * BATCH, BATCH)]], sem_c)

    def _cgroup(i, _):
        for k in range(6):
            pltpu.async_copy(
                b1, acc.at[idx_all.at[pl.ds(r0 + (6 * i + k) * BATCH, BATCH)]],
                sem_c, add=True)
        @pl.when(i > 0)
        def _():
            for k in range(6):
                cnt_desc(6 * (i - 1) + k).wait()
        return 0
    lax.fori_loop(0, CFULL // 6, _cgroup, 0)
    for k in range(6):
        cnt_desc(CFULL - 6 + k).wait()
    pltpu.sync_copy(b1.at[pl.ds(0, CTAIL)],
                    acc.at[idx_all.at[pl.ds(r0 + CFULL * BATCH, CTAIL)]],
                    add=True)
    plsc.subcore_barrier()

    @pl.when(jnp.logical_not(is_last))
    def _():
        pltpu.sync_copy(acc.at[pl.ds(seg0, SEG_SLICE), pl.ds(0, 16)],
                        cnt_hbm.at[core, pl.ds(seg0, SEG_SLICE)])
    @pl.when(is_last)
    def _():
        pltpu.sync_copy(acc.at[pl.ds(seg0, SEG_SLICE_LAST), pl.ds(0, 16)],
                        cnt_hbm.at[core, pl.ds(seg0, SEG_SLICE_LAST)])

    for j in range(NCHUNK):
        d = NCHUNK * core + j

        _fill(b0, jnp.zeros((16,), jnp.float32))
        _zero_acc()
        plsc.subcore_barrier()

        load_start(0, bufs[0], lsems[0], d)
        load_start(1, bufs[1], lsems[1], d)

        def _step(i, _):
            for c in range(4):
                B = 4 * i + c
                k, k2 = c, (c + 2) % 4
                load_wait(B, bufs[k], lsems[k], d)
                scat_start(B, bufs[k], ssems[k])
                @pl.when(B >= 2)
                def _():
                    scat_desc(B - 2, bufs[k2], ssems[k2]).wait()
                @pl.when(B + 2 < NFULL)
                def _():
                    load_start(B + 2, bufs[k2], lsems[k2], d)
            return 0
        lax.fori_loop(0, NFULL // 4, _step, 0)
        scat_desc(NFULL - 2, bufs[(NFULL - 2) % 4], ssems[(NFULL - 2) % 4]).wait()
        scat_desc(NFULL - 1, bufs[(NFULL - 1) % 4], ssems[(NFULL - 1) % 4]).wait()

        a_t = sub * A_PER_TILE + NFULL * SLABS
        pltpu.sync_copy(x_hbm.at[a_t, d], tail_buf.at[pl.ds(0, 8)])
        pltpu.sync_copy(x_hbm.at[a_t + 1, d], tail_buf.at[pl.ds(8, 8)])
        pltpu.sync_copy(tail_buf,
                        acc.at[idx_all.at[pl.ds(NFULL * BATCH, TAIL)]],
                        add=True)
        plsc.subcore_barrier()

        nslab = jnp.where(is_last, SEG_SLICE_LAST // 8, SEG_SLICE // 8)

        def _co_desc(k):
            return pltpu.make_async_copy(
                acc.at[pl.ds(seg0 + k * 8, 8)],
                sums_hbm.at[sub * (SEG_SLICE // 8) + k, d], sem_c)

        def _co_fire(k, _):
            _co_desc(k).start()
            return 0
        def _co_drain(k, _):
            _co_desc(k).wait()
            return 0
        lax.fori_loop(0, nslab, _co_fire, 0)
        lax.fori_loop(0, nslab, _co_drain, 0)


_pool = functools.partial(
    pl.kernel,
    out_type=[
        jax.ShapeDtypeStruct((NUM_SEGMENTS // 8, IN_DIM // FCHUNK, 8, FCHUNK),
                             jnp.float32),
        jax.ShapeDtypeStruct((2, NUM_SEGMENTS, 16), jnp.float32),
    ],
    mesh=plsc.VectorSubcoreMesh(core_axis_name="c", subcore_axis_name="s"),
    compiler_params=pltpu.CompilerParams(use_tc_tiling_on_sc=False),
    scratch_types=[
        pltpu.VMEM((ROWS_PER_TILE,), jnp.int32),
        pltpu.VMEM((BATCH, FCHUNK), jnp.float32),
        pltpu.VMEM((BATCH, FCHUNK), jnp.float32),
        pltpu.VMEM((BATCH, FCHUNK), jnp.float32),
        pltpu.VMEM((BATCH, FCHUNK), jnp.float32),
        pltpu.VMEM((TAIL, FCHUNK), jnp.float32),
        pltpu.VMEM_SHARED((NUM_SEGMENTS, FCHUNK), jnp.float32),
        pltpu.SemaphoreType.DMA,
        pltpu.SemaphoreType.DMA,
        pltpu.SemaphoreType.DMA,
        pltpu.SemaphoreType.DMA,
        pltpu.SemaphoreType.DMA,
        pltpu.SemaphoreType.DMA,
        pltpu.SemaphoreType.DMA,
        pltpu.SemaphoreType.DMA,
        pltpu.SemaphoreType.DMA,
    ],
)(_pool_body)


def _proj_body(sums_ref, cnt_ref, w_ref, b_ref, out_ref):
    counts = cnt_ref[0][:, 0:1] + cnt_ref[1][:, 0:1]
    pooled = sums_ref[...] / jnp.maximum(counts, 1.0)
    out_ref[...] = lax.dot_general(
        pooled, w_ref[...], (((1,), (1,)), ((), ())),
        preferred_element_type=jnp.float32) + b_ref[...]


def _project(sums, counts16, W, b2):
    bm = 1000
    grid = (NUM_SEGMENTS // bm,)
    return pl.pallas_call(
        _proj_body,
        grid=grid,
        in_specs=[
            pl.BlockSpec((bm, IN_DIM), lambda i: (i, 0)),
            pl.BlockSpec((2, bm, 16), lambda i: (0, i, 0)),
            pl.BlockSpec((OUT_DIM, IN_DIM), lambda i: (0, 0)),
            pl.BlockSpec((1, OUT_DIM), lambda i: (0, 0)),
        ],
        out_specs=pl.BlockSpec((bm, OUT_DIM), lambda i: (i, 0)),
        out_shape=jax.ShapeDtypeStruct((NUM_SEGMENTS, OUT_DIM), jnp.float32),
    )(sums, counts16, W, b2)


def kernel(io_concat_embed, scatter_idx, W, b):
    idx = scatter_idx.astype(jnp.int32)
    x4 = jnp.transpose(
        jnp.reshape(io_concat_embed,
                    (N_ROWS // 8, 8, IN_DIM // FCHUNK, FCHUNK)),
        (0, 2, 1, 3))
    sums4, counts16 = _pool(x4, idx)
    sums = jnp.reshape(jnp.transpose(sums4, (0, 2, 1, 3)),
                       (NUM_SEGMENTS, IN_DIM))
    return _project(sums, counts16, W, b.reshape(1, OUT_DIM))
